# Initial kernel scaffold; baseline (speedup 1.0000x reference)
import jax
import jax.numpy as jnp
from jax.experimental import pallas as pl


def kernel(heatmap_heads, offset_heads, wh_heads):
    raise NotImplementedError("write your pallas kernel here")


# speedup vs baseline: 10.7200x; 10.7200x over previous
import jax
import jax.numpy as jnp
from jax import lax
from jax.experimental import pallas as pl
from jax.experimental.pallas import tpu as pltpu

B, C, H, W = 16, 80, 128, 128
K = 100
BIG = 1 << 30


def _decode_kernel(hm_ref, off_ref, wh_ref, scores_ref, classes_ref, bbox_ref,
                   masked_ref):
    x = hm_ref[0]
    s = jax.nn.sigmoid(x)

    riota = lax.broadcasted_iota(jnp.int32, (C, H, W), 1)
    ciota = lax.broadcasted_iota(jnp.int32, (C, H, W), 2)
    neg = jnp.float32(-1.0)

    h = jnp.maximum(s, jnp.where(ciota > 0, jnp.roll(s, 1, axis=2), neg))
    h = jnp.maximum(h, jnp.where(ciota < W - 1, jnp.roll(s, -1, axis=2), neg))
    v = jnp.maximum(h, jnp.where(riota > 0, jnp.roll(h, 1, axis=1), neg))
    v = jnp.maximum(v, jnp.where(riota < H - 1, jnp.roll(h, -1, axis=1), neg))

    masked = jnp.where(v == s, s, 0.0)
    masked_ref[...] = masked
    rowmax0 = masked.max(axis=2)

    fi = lax.broadcasted_iota(jnp.int32, (C, H), 0) * H + \
        lax.broadcasted_iota(jnp.int32, (C, H), 1)
    fi8 = lax.broadcasted_iota(jnp.int32, (8, W), 0) * W + \
        lax.broadcasted_iota(jnp.int32, (8, W), 1)
    ri8 = lax.broadcasted_iota(jnp.int32, (8, W), 0)
    lane = lax.broadcasted_iota(jnp.int32, (1, 128), 1)

    def body(k, carry):
        rowmax, score_v, cls_v, idx_v = carry
        m = jnp.max(rowmax)
        bi = jnp.min(jnp.where(rowmax == m, fi, BIG))
        c = bi // H
        r = bi - c * H
        rt = (r // 8) * 8
        blk = masked_ref[c, pl.ds(rt, 8), :]
        e = jnp.min(jnp.where(blk == m, fi8, BIG))
        spat = rt * W + e
        onek = lane == k
        score_v = jnp.where(onek, m, score_v)
        cls_v = jnp.where(onek, c, cls_v)
        idx_v = jnp.where(onek, spat, idx_v)
        blk2 = jnp.where(fi8 == e, neg, blk)
        masked_ref[c, pl.ds(rt, 8), :] = blk2
        nm = jnp.max(jnp.where(ri8 == e // W, blk2, neg))
        rowmax = jnp.where(fi == bi, nm, rowmax)
        return rowmax, score_v, cls_v, idx_v

    init = (rowmax0, jnp.zeros((1, 128), jnp.float32),
            jnp.zeros((1, 128), jnp.int32), jnp.zeros((1, 128), jnp.int32))
    _, score_v, cls_v, idx_v = lax.fori_loop(0, K, body, init)

    y_v = idx_v // W
    x_v = idx_v % W
    iota_s = lax.broadcasted_iota(jnp.int32, (128, 128), 0)
    oy = (iota_s == y_v).astype(jnp.float32)
    ox = (iota_s == x_v).astype(jnp.float32)

    def gather2d(plane):
        tmp = lax.dot_general(plane, oy, (((0,), (0,)), ((), ())),
                              preferred_element_type=jnp.float32)
        return jnp.sum(tmp * ox, axis=0, keepdims=True)

    g_ox = gather2d(off_ref[0, 0])
    g_oy = gather2d(off_ref[0, 1])
    g_wx = gather2d(wh_ref[0, 0])
    g_wy = gather2d(wh_ref[0, 1])

    xs = x_v.astype(jnp.float32) + g_ox
    ys = y_v.astype(jnp.float32) + g_oy
    x1 = jnp.maximum((xs - g_wx / 2.0) * 4.0, 0.0)
    y1 = jnp.maximum((ys - g_wy / 2.0) * 4.0, 0.0)
    x2 = jnp.minimum((xs + g_wx / 2.0) * 4.0, 511.0)
    y2 = jnp.minimum((ys + g_wy / 2.0) * 4.0, 511.0)

    scores_ref[0] = score_v
    classes_ref[0] = cls_v
    z = jnp.zeros((4, 128), jnp.float32)
    bbox_ref[0] = jnp.concatenate([x1, y1, x2, y2, z], axis=0)


def kernel(heatmap_heads, offset_heads, wh_heads):
    scores, classes, bbox = pl.pallas_call(
        _decode_kernel,
        grid=(B,),
        in_specs=[
            pl.BlockSpec((1, C, H, W), lambda i: (i, 0, 0, 0)),
            pl.BlockSpec((1, 2, H, W), lambda i: (i, 0, 0, 0)),
            pl.BlockSpec((1, 2, H, W), lambda i: (i, 0, 0, 0)),
        ],
        out_specs=[
            pl.BlockSpec((1, 1, 128), lambda i: (i, 0, 0)),
            pl.BlockSpec((1, 1, 128), lambda i: (i, 0, 0)),
            pl.BlockSpec((1, 8, 128), lambda i: (i, 0, 0)),
        ],
        out_shape=[
            jax.ShapeDtypeStruct((B, 1, 128), jnp.float32),
            jax.ShapeDtypeStruct((B, 1, 128), jnp.int32),
            jax.ShapeDtypeStruct((B, 8, 128), jnp.float32),
        ],
        scratch_shapes=[pltpu.VMEM((C, H, W), jnp.float32)],
    )(heatmap_heads, offset_heads, wh_heads)
    return (scores[:, 0, :K], classes[:, 0, :K],
            bbox[:, :4, :K].transpose(0, 2, 1))


# speedup vs baseline: 10.7233x; 1.0003x over previous
import jax
import jax.numpy as jnp
from jax import lax
from jax.experimental import pallas as pl
from jax.experimental.pallas import tpu as pltpu

B, C, H, W = 16, 80, 128, 128
K = 100
BIG = 1 << 30


def _decode_kernel(hm_ref, off_ref, wh_ref, scores_ref, classes_ref, bbox_ref,
                   masked_ref):
    x = hm_ref[0]
    s = jax.nn.sigmoid(x)

    riota = lax.broadcasted_iota(jnp.int32, (C, H, W), 1)
    ciota = lax.broadcasted_iota(jnp.int32, (C, H, W), 2)
    neg = jnp.float32(-1.0)

    h = jnp.maximum(s, jnp.where(ciota > 0, jnp.roll(s, 1, axis=2), neg))
    h = jnp.maximum(h, jnp.where(ciota < W - 1, jnp.roll(s, -1, axis=2), neg))
    v = jnp.maximum(h, jnp.where(riota > 0, jnp.roll(h, 1, axis=1), neg))
    v = jnp.maximum(v, jnp.where(riota < H - 1, jnp.roll(h, -1, axis=1), neg))

    masked = jnp.where(v == s, s, 0.0)
    masked_ref[...] = masked
    rowmax0 = masked.max(axis=2)

    fi = lax.broadcasted_iota(jnp.int32, (C, H), 0) * H + \
        lax.broadcasted_iota(jnp.int32, (C, H), 1)
    fi8 = lax.broadcasted_iota(jnp.int32, (8, W), 0) * W + \
        lax.broadcasted_iota(jnp.int32, (8, W), 1)
    ri8 = lax.broadcasted_iota(jnp.int32, (8, W), 0)
    lane = lax.broadcasted_iota(jnp.int32, (1, 128), 1)

    def body(k, carry):
        rowmax, score_v, cls_v, idx_v = carry
        m = jnp.max(rowmax)
        bi = jnp.min(jnp.where(rowmax == m, fi, BIG))
        c = bi // H
        r = bi - c * H
        rt = (r // 8) * 8
        blk = masked_ref[c, pl.ds(rt, 8), :]
        e = jnp.min(jnp.where(blk == m, fi8, BIG))
        spat = rt * W + e
        onek = lane == k
        score_v = jnp.where(onek, m, score_v)
        cls_v = jnp.where(onek, c, cls_v)
        idx_v = jnp.where(onek, spat, idx_v)
        blk2 = jnp.where(fi8 == e, neg, blk)
        masked_ref[c, pl.ds(rt, 8), :] = blk2
        nm = jnp.max(jnp.where(ri8 == e // W, blk2, neg))
        rowmax = jnp.where(fi == bi, nm, rowmax)
        return rowmax, score_v, cls_v, idx_v

    init = (rowmax0, jnp.zeros((1, 128), jnp.float32),
            jnp.zeros((1, 128), jnp.int32), jnp.zeros((1, 128), jnp.int32))
    _, score_v, cls_v, idx_v = lax.fori_loop(0, K, body, init)

    y_v = idx_v // W
    x_v = idx_v % W
    iota_s = lax.broadcasted_iota(jnp.int32, (128, 128), 0)
    oy = (iota_s == y_v).astype(jnp.float32)
    ox = (iota_s == x_v).astype(jnp.float32)

    def gather2d(plane):
        tmp = lax.dot_general(plane, oy, (((0,), (0,)), ((), ())),
                              preferred_element_type=jnp.float32)
        return jnp.sum(tmp * ox, axis=0, keepdims=True)

    g_ox = gather2d(off_ref[0, 0])
    g_oy = gather2d(off_ref[0, 1])
    g_wx = gather2d(wh_ref[0, 0])
    g_wy = gather2d(wh_ref[0, 1])

    xs = x_v.astype(jnp.float32) + g_ox
    ys = y_v.astype(jnp.float32) + g_oy
    x1 = jnp.maximum((xs - g_wx / 2.0) * 4.0, 0.0)
    y1 = jnp.maximum((ys - g_wy / 2.0) * 4.0, 0.0)
    x2 = jnp.minimum((xs + g_wx / 2.0) * 4.0, 511.0)
    y2 = jnp.minimum((ys + g_wy / 2.0) * 4.0, 511.0)

    scores_ref[0] = score_v
    classes_ref[0] = cls_v
    z = jnp.zeros((4, 128), jnp.float32)
    bbox_ref[0] = jnp.concatenate([x1, y1, x2, y2, z], axis=0)


def kernel(heatmap_heads, offset_heads, wh_heads):
    scores, classes, bbox = pl.pallas_call(
        _decode_kernel,
        grid=(B,),
        in_specs=[
            pl.BlockSpec((1, C, H, W), lambda i: (i, 0, 0, 0)),
            pl.BlockSpec((1, 2, H, W), lambda i: (i, 0, 0, 0)),
            pl.BlockSpec((1, 2, H, W), lambda i: (i, 0, 0, 0)),
        ],
        out_specs=[
            pl.BlockSpec((1, 1, 128), lambda i: (i, 0, 0)),
            pl.BlockSpec((1, 1, 128), lambda i: (i, 0, 0)),
            pl.BlockSpec((1, 8, 128), lambda i: (i, 0, 0)),
        ],
        out_shape=[
            jax.ShapeDtypeStruct((B, 1, 128), jnp.float32),
            jax.ShapeDtypeStruct((B, 1, 128), jnp.int32),
            jax.ShapeDtypeStruct((B, 8, 128), jnp.float32),
        ],
        scratch_shapes=[pltpu.VMEM((C, H, W), jnp.float32)],
        compiler_params=pltpu.CompilerParams(
            dimension_semantics=("parallel",)),
    )(heatmap_heads, offset_heads, wh_heads)
    return (scores[:, 0, :K], classes[:, 0, :K],
            bbox[:, :4, :K].transpose(0, 2, 1))
